# Initial kernel scaffold; baseline (speedup 1.0000x reference)
import jax
import jax.numpy as jnp
from jax.experimental import pallas as pl


def kernel(x, edge_index, batch, emb, Wrel1, brel1, Wroot1, p1, Wrel2, brel2, Wroot2, p2, Wrel3, brel3, Wroot3, p3, W1, lb1, W2, lb2, W3, lb3, g1, be1, g2, be2):
    raise NotImplementedError("write your pallas kernel here")


# speedup vs baseline: 15.1403x; 15.1403x over previous
import functools
import numpy as np
import jax
import jax.numpy as jnp
from jax import lax
from jax.experimental import pallas as pl
from jax.experimental.pallas import tpu as pltpu
from jax.experimental.pallas import tpu_sc as plsc

_N = 10000
_E = 320000
_D = 128
_NG = 256
_NP = 10240
_NT = _NP // 512
_EPS = 1e-5
_NW = 32
_EPW = _E // _NW
_ECH = 80
_NECH = _EPW // _ECH
_ROWS_PER_TILE = _NP // 16
_NEG = -1e30

_HI = jax.lax.Precision.HIGHEST


def _sc_gather(table, idx3, nch, dt):
  mesh = plsc.VectorSubcoreMesh(core_axis_name="c", subcore_axis_name="s")
  b = idx3.shape[0] * idx3.shape[1] * idx3.shape[2]

  @functools.partial(
      pl.kernel, mesh=mesh,
      out_type=jax.ShapeDtypeStruct((b, dt), jnp.float32),
      scratch_types=[
          pltpu.VMEM((nch, _ECH), jnp.int32),
          pltpu.VMEM((_ECH, dt), jnp.float32),
          pltpu.SemaphoreType.DMA,
      ],
  )
  def k(table_hbm, idx_hbm, out_hbm, idx_v, rows_v, sem):
    cid = lax.axis_index("c")
    sid = lax.axis_index("s")
    wid = cid * 16 + sid
    pltpu.sync_copy(idx_hbm.at[wid], idx_v)

    def chunk(j, _):
      pltpu.async_copy(table_hbm.at[idx_v.at[j]], rows_v, sem).wait()
      pltpu.sync_copy(rows_v, out_hbm.at[pl.ds(wid * (nch * _ECH) + j * _ECH,
                                               _ECH)])
      return 0

    lax.fori_loop(0, nch, chunk, 0)

  return k(table, idx3)


_NH = _NP // 2
_NECH2 = _E // 16 // _ECH
_RPT = _NH // 16


def _sc_edge(m, src3, dst3):
  mesh = plsc.VectorSubcoreMesh(core_axis_name="c", subcore_axis_name="s")

  @functools.partial(
      pl.kernel, mesh=mesh,
      out_type=jax.ShapeDtypeStruct((_NP, _D), jnp.float32),
      scratch_types=[
          pltpu.VMEM((_NECH2, _ECH), jnp.int32),
          pltpu.VMEM((_NECH2, _ECH), jnp.int32),
          pltpu.VMEM((_ECH, _D), jnp.float32),
          pltpu.VMEM((_ECH, _D), jnp.float32),
          pltpu.VMEM_SHARED((_NH + 8, _D), jnp.float32),
          pltpu.SemaphoreType.DMA,
      ],
  )
  def k(m_hbm, src_hbm, dst_hbm, out_hbm, src_v, dst_v, rows_v, zbuf, agg_sh,
        sem):
    cid = lax.axis_index("c")
    sid = lax.axis_index("s")
    pltpu.sync_copy(src_hbm.at[sid], src_v)
    pltpu.sync_copy(dst_hbm.at[sid], dst_v)

    off = cid * _NH

    def orow(r, _):
      for c in range(_ECH // 16):
        d = dst_v[r, pl.ds(c * 16, 16)] - off
        d = jnp.where((d >= 0) & (d < _NH), d, _NH)
        dst_v[r, pl.ds(c * 16, 16)] = d
      return 0
    lax.fori_loop(0, _NECH2, orow, 0)

    def zrow(r, _):
      for c in range(_D // 16):
        zbuf[r, pl.ds(c * 16, 16)] = jnp.zeros((16,), jnp.float32)
      return 0
    lax.fori_loop(0, _ECH, zrow, 0)
    for b in range(_RPT // _ECH):
      pltpu.sync_copy(zbuf, agg_sh.at[pl.ds(sid * _RPT + b * _ECH, _ECH)])
    plsc.subcore_barrier()

    def chunk(j, _):
      pltpu.async_copy(m_hbm.at[src_v.at[j]], rows_v, sem).wait()
      pltpu.sync_copy(rows_v, agg_sh.at[dst_v.at[j]], add=True)
      return 0

    lax.fori_loop(0, _NECH2, chunk, 0)
    plsc.subcore_barrier()
    pltpu.sync_copy(
        agg_sh.at[pl.ds(sid * _RPT, _RPT)],
        out_hbm.at[pl.ds(cid * _NH + sid * _RPT, _RPT)])

  return k(m, src3, dst3)


def _tc_stage_pre(agg, h_in, wrel, wroot, brel, p, valid_col, batch_col):
  def body(agg_ref, hin_ref, wrel_ref, wroot_ref, brel_ref, p_ref, v_ref,
           b_ref, h_ref, s_ref, k_ref):
    i = pl.program_id(0)
    acc = (jnp.dot(agg_ref[...], wrel_ref[...])
           + jnp.dot(hin_ref[...], wroot_ref[...]) + brel_ref[...])
    h = jnp.where(acc > 0, acc, 0.01 * acc)
    h = h * v_ref[...]
    h_ref[...] = h
    pn = p_ref[...]
    s_ref[...] = jnp.dot(h, pn) / jnp.sqrt(jnp.sum(pn * pn))
    gids = lax.broadcasted_iota(jnp.int32, (1, _NG), 1)
    onehot = jnp.where((b_ref[...] == gids) & (v_ref[...] > 0), 1.0, 0.0)
    cnt = jnp.sum(onehot, axis=0, keepdims=True)

    @pl.when(i == 0)
    def _():
      k_ref[...] = cnt

    @pl.when(i > 0)
    def _():
      k_ref[...] = k_ref[...] + cnt

    @pl.when(i == _NT - 1)
    def _():
      k_ref[...] = jnp.ceil(0.8 * k_ref[...])

  return pl.pallas_call(
      body,
      grid=(_NT,),
      in_specs=[
          pl.BlockSpec((512, _D), lambda i: (i, 0)),
          pl.BlockSpec((512, _D), lambda i: (i, 0)),
          pl.BlockSpec((_D, _D), lambda i: (0, 0)),
          pl.BlockSpec((_D, _D), lambda i: (0, 0)),
          pl.BlockSpec((1, _D), lambda i: (0, 0)),
          pl.BlockSpec((_D, 1), lambda i: (0, 0)),
          pl.BlockSpec((512, 1), lambda i: (i, 0)),
          pl.BlockSpec((512, 1), lambda i: (i, 0)),
      ],
      out_specs=[
          pl.BlockSpec((512, _D), lambda i: (i, 0)),
          pl.BlockSpec((512, 1), lambda i: (i, 0)),
          pl.BlockSpec((1, _NG), lambda i: (0, 0)),
      ],
      out_shape=[
          jax.ShapeDtypeStruct((_NP, _D), jnp.float32),
          jax.ShapeDtypeStruct((_NP, 1), jnp.float32),
          jax.ShapeDtypeStruct((1, _NG), jnp.float32),
      ],
  )(agg, h_in, wrel, wroot, brel, p, valid_col, batch_col)


def _tc_topk_gate(score_col, score_row, batch_col, batch_row, valid_row,
                  valid_col, k_col, hpre):
  def body(sc_ref, sr_ref, bc_ref, br_ref, vr_ref, vc_ref, k_ref, h_ref,
           kept_ref, hn_ref, rank_ref):
    i = pl.program_id(0)
    si = sc_ref[pl.ds(i * 512, 512), :]
    bi = bc_ref[pl.ds(i * 512, 512), :]
    vi = vc_ref[...]
    ui = lax.broadcasted_iota(jnp.int32, (512, 1), 0) + i * 512
    bmin_i = jnp.min(bi)
    bmax_i = jnp.max(bi)
    rank_ref[...] = jnp.zeros((512, 1), jnp.float32)

    def jstep(j, _):
      bj = br_ref[pl.ds(j, 1), :]
      bmin_j = jnp.min(bj)
      bmax_j = jnp.max(bj)

      @pl.when((bmin_j <= bmax_i) & (bmin_i <= bmax_j))
      def _():
        sj = sr_ref[pl.ds(j, 1), :]
        vj = vr_ref[pl.ds(j, 1), :]
        uj = lax.broadcasted_iota(jnp.int32, (1, 512), 1) + j * 512
        same = (bj == bi) & (vj > 0)
        beats = (sj > si) | ((sj == si) & (uj < ui))
        c = jnp.sum(jnp.where(same & beats, 1.0, 0.0), axis=1, keepdims=True)
        rank_ref[...] = rank_ref[...] + c

      return 0

    lax.fori_loop(0, _NT, jstep, 0)
    gids = lax.broadcasted_iota(jnp.int32, (1, _NG), 1)
    onehot = jnp.where(bi == gids, 1.0, 0.0)
    kk = jnp.dot(onehot, k_ref[...], precision=_HI)
    kept = jnp.where((vi > 0) & (rank_ref[...] < kk), 1.0, 0.0)
    kept_ref[...] = kept
    th = jnp.tanh(si)
    hn_ref[...] = h_ref[...] * th * kept

  return pl.pallas_call(
      body,
      grid=(_NT,),
      in_specs=[
          pl.BlockSpec((_NP, 1), lambda i: (0, 0)),
          pl.BlockSpec((_NT, 512), lambda i: (0, 0)),
          pl.BlockSpec((_NP, 1), lambda i: (0, 0)),
          pl.BlockSpec((_NT, 512), lambda i: (0, 0)),
          pl.BlockSpec((_NT, 512), lambda i: (0, 0)),
          pl.BlockSpec((512, 1), lambda i: (i, 0)),
          pl.BlockSpec((_NG, 1), lambda i: (0, 0)),
          pl.BlockSpec((512, _D), lambda i: (i, 0)),
      ],
      out_specs=[
          pl.BlockSpec((512, 1), lambda i: (i, 0)),
          pl.BlockSpec((512, _D), lambda i: (i, 0)),
      ],
      out_shape=[
          jax.ShapeDtypeStruct((_NP, 1), jnp.float32),
          jax.ShapeDtypeStruct((_NP, _D), jnp.float32),
      ],
      scratch_shapes=[pltpu.VMEM((512, 1), jnp.float32)],
  )(score_col, score_row, batch_col, batch_row, valid_row, valid_col, k_col,
    hpre)


def _tc_readout(h, batch_col, batch_row, kept_col, kept_row, k_col):
  def body(h_ref, bc_ref, br_ref, kc_ref, kr_ref, k_ref, mx_ref, sm_ref):
    hv = h_ref[...]
    bc = bc_ref[...]
    val = jnp.where(kc_ref[...] > 0, hv, _NEG)
    rowid = lax.broadcasted_iota(jnp.int32, (_NP, 1), 0)
    for s in range(14):
      sh = 1 << s
      vsh = pltpu.roll(val, sh, 0)
      bsh = pltpu.roll(bc, sh, 0)
      ok = (rowid >= sh) & (bsh == bc)
      val = jnp.where(ok, jnp.maximum(val, vsh), val)
    br = br_ref[...]
    bnext = pltpu.roll(br, _NP - 1, 1)
    isend = jnp.where(
        (br != bnext) | (lax.broadcasted_iota(jnp.int32, (1, _NP), 1)
                         == _NP - 1), 1.0, 0.0)
    gids = lax.broadcasted_iota(jnp.int32, (_NG, 1), 0)
    kr = kr_ref[...]
    mx = jnp.zeros((_NG, _D), jnp.float32)
    sm = jnp.zeros((_NG, _D), jnp.float32)
    for t in range(_NT):
      brt = br[:, t * 512:(t + 1) * 512]
      oh = jnp.where(gids == brt, 1.0, 0.0)
      et = oh * isend[:, t * 512:(t + 1) * 512]
      st = oh * kr[:, t * 512:(t + 1) * 512]
      mx = mx + jnp.dot(et, val[t * 512:(t + 1) * 512, :], precision=_HI)
      sm = sm + jnp.dot(st, hv[t * 512:(t + 1) * 512, :], precision=_HI)
    mx_ref[...] = mx
    sm_ref[...] = sm / jnp.maximum(k_ref[...], 1.0)

  return pl.pallas_call(
      body,
      out_shape=[
          jax.ShapeDtypeStruct((_NG, _D), jnp.float32),
          jax.ShapeDtypeStruct((_NG, _D), jnp.float32),
      ],
  )(h, batch_col, batch_row, kept_col, kept_row, k_col)


def _tc_mlp(mxs, sms, W1, lb1, g1, be1, W2, lb2, g2, be2, W3, lb3):
  sc = 1.0 / np.sqrt(1.0 + _EPS)

  def body(mx_ref, sm_ref, w1a_ref, w1b_ref, lb1_ref, g1_ref, be1_ref,
           w2_ref, lb2_ref, g2_ref, be2_ref, w3_ref, lb3_ref, o_ref):
    z = (jnp.dot(mx_ref[...], w1a_ref[...], precision=_HI)
         + jnp.dot(sm_ref[...], w1b_ref[...], precision=_HI) + lb1_ref[...])
    z = jnp.where(z > 0, z, 0.01 * z)
    z = g1_ref[...] * z * sc + be1_ref[...]
    z = jnp.dot(z, w2_ref[...], precision=_HI) + lb2_ref[...]
    z = jnp.where(z > 0, z, 0.01 * z)
    z = g2_ref[...] * z * sc + be2_ref[...]
    o_ref[...] = jnp.dot(z, w3_ref[...], precision=_HI) + lb3_ref[...]

  return pl.pallas_call(
      body,
      out_shape=jax.ShapeDtypeStruct((_NG, 1), jnp.float32),
  )(mxs, sms, W1[:_D], W1[_D:], lb1.reshape(1, -1), g1.reshape(1, -1),
    be1.reshape(1, -1), W2, lb2.reshape(1, -1), g2.reshape(1, -1),
    be2.reshape(1, -1), W3, lb3.reshape(1, 1))


def kernel(x, edge_index, batch, emb, Wrel1, brel1, Wroot1, p1, Wrel2, brel2,
           Wroot2, p2, Wrel3, brel3, Wroot3, p3, W1, lb1, W2, lb2, W3, lb3,
           g1, be1, g2, be2):
  xpad = jnp.concatenate([x[:, 0].astype(jnp.int32),
                          jnp.zeros((_NP - _N,), jnp.int32)])
  x3 = xpad.reshape(_NW, _NP // _NW // _ECH, _ECH)
  src3 = edge_index[0].astype(jnp.int32).reshape(16, _NECH2, _ECH)
  dst3 = edge_index[1].astype(jnp.int32).reshape(16, _NECH2, _ECH)
  bpad = jnp.concatenate([batch.astype(jnp.int32),
                          jnp.full((_NP - _N,), _NG, jnp.int32)])
  batch_col = bpad.reshape(_NP, 1)
  batch_row = bpad.reshape(1, _NP)
  batch_r2 = bpad.reshape(_NT, 512)
  valid = jnp.concatenate([jnp.ones((_N,), jnp.float32),
                           jnp.zeros((_NP - _N,), jnp.float32)])
  emb_p = jnp.concatenate([emb, jnp.zeros((7, _D), jnp.float32)], axis=0)

  stages = [(Wrel1, brel1, Wroot1, p1), (Wrel2, brel2, Wroot2, p2),
            (Wrel3, brel3, Wroot3, p3)]

  hn = _sc_gather(emb_p, x3, _NP // _NW // _ECH, _D)

  valid_col = valid.reshape(_NP, 1)
  valid_r2 = valid.reshape(_NT, 512)
  mxs = None
  sms = None
  for l in range(3):
    wrel, brel, wroot, p = stages[l]
    agg = _sc_edge(hn, src3, dst3)
    hpre, score, kvec = _tc_stage_pre(agg, hn, wrel, wroot,
                                      brel.reshape(1, _D), p.reshape(_D, 1),
                                      valid_col, batch_col)
    k_col = kvec.reshape(_NG, 1)
    kept, hn = _tc_topk_gate(score, score.reshape(_NT, 512), batch_col,
                             batch_r2, valid_r2, valid_col, k_col, hpre)
    mx, sm = _tc_readout(hn, batch_col, batch_row, kept,
                         kept.reshape(1, _NP), k_col)
    mxs = mx if mxs is None else mxs + mx
    sms = sm if sms is None else sms + sm
    valid_col = kept
    valid_r2 = kept.reshape(_NT, 512)

  out = _tc_mlp(mxs, sms, W1, lb1, g1, be1, W2, lb2, g2, be2, W3, lb3)
  return out[:, 0]
